# baseline (speedup 1.0000x reference)
import functools

import jax
import jax.numpy as jnp
from jax import lax
from jax.experimental import pallas as pl
from jax.experimental.pallas import tpu as pltpu
from jax.experimental.pallas import tpu_sc as plsc

N = 10000
E = 320000
D = 128
DH = 64
DG = 16

NC = 2
NS = 16
B = 80
EPT = E // NS
NB = EPT // B
NP = 10240
S = 6
F = 4

_ROWBLK = 1000
_GRID = N // _ROWBLK


def _make_sc_agg(with_deg):
  mesh = plsc.VectorSubcoreMesh(
      core_axis_name="c", subcore_axis_name="s", num_cores=NC, num_subcores=NS)

  out_type = [jax.ShapeDtypeStruct((NC, NP, DH), jnp.float32)]
  scratch = [
      pltpu.VMEM((NB, B), jnp.int32),
      pltpu.VMEM((NB, B), jnp.int32),
      [pltpu.VMEM((B, DH), jnp.float32) for _ in range(S)],
      [pltpu.SemaphoreType.DMA for _ in range(S)],
      [pltpu.SemaphoreType.DMA for _ in range(S)],
      pltpu.VMEM((64, DH), jnp.float32),
      pltpu.VMEM_SHARED((NP, DH), jnp.float32),
  ]
  if with_deg:
    out_type.append(jax.ShapeDtypeStruct((NC, NP, DG), jnp.float32))
    scratch += [
        pltpu.VMEM((B, DG), jnp.float32),
        pltpu.VMEM((64, DG), jnp.float32),
        pltpu.VMEM_SHARED((NP, DG), jnp.float32),
    ]

  @functools.partial(
      pl.kernel,
      out_type=tuple(out_type) if with_deg else out_type[0],
      mesh=mesh,
      compiler_params=pltpu.CompilerParams(use_tc_tiling_on_sc=False),
      scratch_types=scratch,
  )
  def agg(*refs):
    if with_deg:
      (h2_hbm, eidx_hbm, ones_hbm, out_hbm, outdeg_hbm,
       src_v, dst_v, rows, gsems, ssems, zbuf, acc,
       ones_v, zdeg, degacc) = refs
    else:
      (h2_hbm, eidx_hbm, out_hbm,
       src_v, dst_v, rows, gsems, ssems, zbuf, acc) = refs

    c = lax.axis_index("c")
    s = lax.axis_index("s")

    z16 = jnp.zeros((16,), jnp.float32)

    def zrow(i, carry):
      for j in range(DH // 16):
        zbuf[i, pl.ds(j * 16, 16)] = z16
      if with_deg:
        zdeg[i, pl.ds(0, 16)] = z16
      return carry

    lax.fori_loop(0, 64, zrow, 0)
    for j in range(10):
      r = s * 640 + j * 64
      pltpu.sync_copy(zbuf, acc.at[pl.ds(r, 64)])
      if with_deg:
        pltpu.sync_copy(zdeg, degacc.at[pl.ds(r, 64)])
    plsc.subcore_barrier()

    pltpu.sync_copy(eidx_hbm.at[0, s], src_v)
    pltpu.sync_copy(eidx_hbm.at[1, s], dst_v)
    if with_deg:
      pltpu.sync_copy(ones_hbm, ones_v)

    def tx(i, carry):
      for j in range(B // 16):
        sl = (i, pl.ds(j * 16, 16))
        src_v[sl] = src_v[sl] * 2 + c
      return carry

    lax.fori_loop(0, NB, tx, 0)

    def visit(i, r, guard_ssem):
      r2 = (r + F) % S
      wait_sc = lambda: pltpu.make_async_copy(
          rows[r2], acc.at[dst_v.at[i + F - S]], ssems[r2]).wait()
      if guard_ssem:
        pl.when(i + F - S >= 0)(wait_sc)
      else:
        wait_sc()
      pltpu.async_copy(h2_hbm.at[src_v.at[i + F]], rows[r2], gsems[r2])
      pltpu.make_async_copy(h2_hbm.at[src_v.at[i]], rows[r], gsems[r]).wait()
      pltpu.async_copy(rows[r], acc.at[dst_v.at[i]], ssems[r], add=True)

    for j in range(F):
      pltpu.async_copy(h2_hbm.at[src_v.at[j]], rows[j], gsems[j])

    nloop = (NB - F) // S

    def body(g, carry):
      for r in range(S):
        visit(S * g + r, r, guard_ssem=True)
      return carry

    lax.fori_loop(0, nloop, body, 0)

    for i in range(S * nloop, NB):
      r = i % S
      pltpu.make_async_copy(h2_hbm.at[src_v.at[i]], rows[r], gsems[r]).wait()
      pltpu.async_copy(rows[r], acc.at[dst_v.at[i]], ssems[r], add=True)

    for j in range(S * nloop + F - S, NB):
      rj = j % S
      pltpu.make_async_copy(rows[rj], acc.at[dst_v.at[j]], ssems[rj]).wait()

    if with_deg:
      def dbody(i, carry):
        pltpu.sync_copy(ones_v, degacc.at[dst_v.at[i]], add=True)
        return carry

      lax.fori_loop(c * (NB // 2), (c + 1) * (NB // 2), dbody, 0)

    plsc.subcore_barrier()

    for j in range(10):
      r = s * 640 + j * 64
      pltpu.sync_copy(acc.at[pl.ds(r, 64)], out_hbm.at[c, pl.ds(r, 64)])
      if with_deg:
        pltpu.sync_copy(degacc.at[pl.ds(r, 64)], outdeg_hbm.at[c, pl.ds(r, 64)])

  return agg


_sc_agg_deg = _make_sc_agg(True)
_sc_agg = _make_sc_agg(False)


def _tc_pre(x_ref, w_ref, b_ref, xs_ref, xn_ref):
  y = jnp.dot(x_ref[...], w_ref[...], preferred_element_type=jnp.float32)
  xs_ref[...] = y[:, :D] + b_ref[...]
  xn_ref[...] = y[:, D:]


def _tc_mid(xs_ref, p0_ref, p1_ref, pd0_ref, pd1_ref, w_ref, b_ref,
            xs2_ref, xn2_ref, rdeg_ref):
  deg = pd0_ref[0][:, :1] + pd1_ref[0][:, :1]
  rdeg = 1.0 / jnp.maximum(deg, 1.0)
  neigh = jnp.concatenate([p0_ref[0], p1_ref[0]], axis=1)
  h1 = jnp.maximum(xs_ref[...] + neigh * rdeg, 0.0)
  y = jnp.dot(h1, w_ref[...], preferred_element_type=jnp.float32)
  xs2_ref[...] = y[:, :D] + b_ref[...]
  xn2_ref[...] = y[:, D:]
  rdeg_ref[...] = rdeg


def _tc_post(xs2_ref, q0_ref, q1_ref, rdeg_ref, out_ref):
  neigh = jnp.concatenate([q0_ref[0], q1_ref[0]], axis=1)
  out_ref[...] = xs2_ref[...] + neigh * rdeg_ref[...]


def _row_spec(cols):
  return pl.BlockSpec((_ROWBLK, cols), lambda i: (i, 0))


_W_SPEC = pl.BlockSpec((D, 2 * D), lambda i: (0, 0))
_B_SPEC = pl.BlockSpec((1, D), lambda i: (0, 0))


def _part_spec(cols, core):
  return pl.BlockSpec((1, _ROWBLK, cols), lambda i, c=core: (c, i, 0))


def kernel(x, edge_index, W_self1, W_neigh1, b1, W_self2, W_neigh2, b2):
  eidx = edge_index.reshape(2, NS, NB, B)
  w1 = jnp.concatenate([W_self1, W_neigh1], axis=1)
  w2 = jnp.concatenate([W_self2, W_neigh2], axis=1)
  b1r = b1.reshape(1, D)
  b2r = b2.reshape(1, D)
  ones16 = jnp.ones((B, DG), jnp.float32)

  xs1, xn1 = pl.pallas_call(
      _tc_pre,
      grid=(_GRID,),
      in_specs=[_row_spec(D), _W_SPEC, _B_SPEC],
      out_specs=[_row_spec(D), _row_spec(D)],
      out_shape=[
          jax.ShapeDtypeStruct((N, D), jnp.float32),
          jax.ShapeDtypeStruct((N, D), jnp.float32),
      ],
  )(x, w1, b1r)

  p, pdeg = _sc_agg_deg(xn1.reshape(2 * N, DH), eidx, ones16)

  xs2, xn2, rdeg = pl.pallas_call(
      _tc_mid,
      grid=(_GRID,),
      in_specs=[_row_spec(D), _part_spec(DH, 0), _part_spec(DH, 1),
                _part_spec(DG, 0), _part_spec(DG, 1), _W_SPEC, _B_SPEC],
      out_specs=[_row_spec(D), _row_spec(D), _row_spec(1)],
      out_shape=[
          jax.ShapeDtypeStruct((N, D), jnp.float32),
          jax.ShapeDtypeStruct((N, D), jnp.float32),
          jax.ShapeDtypeStruct((N, 1), jnp.float32),
      ],
  )(xs1, p, p, pdeg, pdeg, w2, b2r)

  q = _sc_agg(xn2.reshape(2 * N, DH), eidx)

  out = pl.pallas_call(
      _tc_post,
      grid=(_GRID,),
      in_specs=[_row_spec(D), _part_spec(DH, 0), _part_spec(DH, 1),
                _row_spec(1)],
      out_specs=_row_spec(D),
      out_shape=jax.ShapeDtypeStruct((N, D), jnp.float32),
  )(xs2, q, q, rdeg)

  return out


# speedup vs baseline: 1.0427x; 1.0427x over previous
import functools

import jax
import jax.numpy as jnp
from jax import lax
from jax.experimental import pallas as pl
from jax.experimental.pallas import tpu as pltpu
from jax.experimental.pallas import tpu_sc as plsc

N = 10000
E = 320000
D = 128
DH = 64
DG = 16

NC = 2
NS = 16
B = 80
EPT = E // NS
NB = EPT // B
NP = 10240
S = 6
F = 4

_ROWBLK = 2000
_GRID = N // _ROWBLK
_PB = _ROWBLK // 2


def _make_sc_agg(with_deg):
  mesh = plsc.VectorSubcoreMesh(
      core_axis_name="c", subcore_axis_name="s", num_cores=NC, num_subcores=NS)

  out_type = [jax.ShapeDtypeStruct((NC, NP, DH), jnp.float32)]
  scratch = [
      pltpu.VMEM((NB, B), jnp.int32),
      pltpu.VMEM((NB, B), jnp.int32),
      [pltpu.VMEM((B, DH), jnp.float32) for _ in range(S)],
      [pltpu.SemaphoreType.DMA for _ in range(S)],
      [pltpu.SemaphoreType.DMA for _ in range(S)],
      pltpu.VMEM((64, DH), jnp.float32),
      pltpu.VMEM_SHARED((NP, DH), jnp.float32),
  ]
  if with_deg:
    out_type.append(jax.ShapeDtypeStruct((NP, DH), jnp.float32))
    scratch += [
        pltpu.VMEM((B, DG), jnp.float32),
        pltpu.VMEM((64, DG), jnp.float32),
        pltpu.VMEM((64, DG), jnp.float32),
        pltpu.VMEM_SHARED((NP, DG), jnp.float32),
        pltpu.SemaphoreType.DMA,
    ]

  @functools.partial(
      pl.kernel,
      out_type=tuple(out_type) if with_deg else out_type[0],
      mesh=mesh,
      compiler_params=pltpu.CompilerParams(use_tc_tiling_on_sc=False),
      scratch_types=scratch,
  )
  def agg(*refs):
    if with_deg:
      (h2_hbm, eidx_hbm, onesz_hbm, out_hbm, rdeg_hbm,
       src_v, dst_v, rows, gsems, ssems, zbuf, acc,
       ones_v, zdeg_v, dback, degacc, dsem) = refs
    else:
      (h2_hbm, eidx_hbm, out_hbm,
       src_v, dst_v, rows, gsems, ssems, zbuf, acc) = refs

    c = lax.axis_index("c")
    s = lax.axis_index("s")

    z16 = jnp.zeros((16,), jnp.float32)

    def zrow(i, carry):
      for j in range(DH // 16):
        zbuf[i, pl.ds(j * 16, 16)] = z16
      return carry

    lax.fori_loop(0, 64, zrow, 0)
    if with_deg:
      pltpu.sync_copy(onesz_hbm.at[0], ones_v)
      pltpu.sync_copy(onesz_hbm.at[1, pl.ds(0, 64)], zdeg_v)
    for j in range(10):
      r = s * 640 + j * 64
      pltpu.sync_copy(zbuf, acc.at[pl.ds(r, 64)])
      if with_deg:
        pltpu.sync_copy(zdeg_v, degacc.at[pl.ds(r, 64)])
    plsc.subcore_barrier()

    pltpu.sync_copy(eidx_hbm.at[0, s], src_v)
    pltpu.sync_copy(eidx_hbm.at[1, s], dst_v)

    def tx(i, carry):
      for j in range(B // 16):
        sl = (i, pl.ds(j * 16, 16))
        src_v[sl] = src_v[sl] * 2 + c
      return carry

    lax.fori_loop(0, NB, tx, 0)

    def visit(i, r, guard_ssem):
      r2 = (r + F) % S
      wait_sc = lambda: pltpu.make_async_copy(
          rows[r2], acc.at[dst_v.at[i + F - S]], ssems[r2]).wait()
      if guard_ssem:
        pl.when(i + F - S >= 0)(wait_sc)
      else:
        wait_sc()
      pltpu.async_copy(h2_hbm.at[src_v.at[i + F]], rows[r2], gsems[r2])
      pltpu.make_async_copy(h2_hbm.at[src_v.at[i]], rows[r], gsems[r]).wait()
      pltpu.async_copy(rows[r], acc.at[dst_v.at[i]], ssems[r], add=True)
      if with_deg:
        def _fire_deg(i=i):
          pltpu.async_copy(ones_v, degacc.at[dst_v.at[i]], dsem, add=True)

        pl.when(c == 0)(_fire_deg)

    for j in range(F):
      pltpu.async_copy(h2_hbm.at[src_v.at[j]], rows[j], gsems[j])

    nloop = (NB - F) // S

    def body(g, carry):
      for r in range(S):
        visit(S * g + r, r, guard_ssem=True)
      return carry

    lax.fori_loop(0, nloop, body, 0)

    for i in range(S * nloop, NB):
      r = i % S
      pltpu.make_async_copy(h2_hbm.at[src_v.at[i]], rows[r], gsems[r]).wait()
      pltpu.async_copy(rows[r], acc.at[dst_v.at[i]], ssems[r], add=True)
      if with_deg:
        def _fire_deg(i=i):
          pltpu.async_copy(ones_v, degacc.at[dst_v.at[i]], dsem, add=True)

        pl.when(c == 0)(_fire_deg)

    for j in range(S * nloop + F - S, NB):
      rj = j % S
      pltpu.make_async_copy(rows[rj], acc.at[dst_v.at[j]], ssems[rj]).wait()

    if with_deg:
      def ddrain(i, carry):
        pltpu.make_async_copy(ones_v, degacc.at[dst_v.at[i]], dsem).wait()
        return carry

      @pl.when(c == 0)
      def _():
        lax.fori_loop(0, NB, ddrain, 0)

    plsc.subcore_barrier()

    for j in range(10):
      r = s * 640 + j * 64
      pltpu.sync_copy(acc.at[pl.ds(r, 64)], out_hbm.at[c, pl.ds(r, 64)])
      if with_deg:
        @pl.when(c == 0)
        def _(r=r):
          pltpu.sync_copy(degacc.at[pl.ds(r, 64)], dback)

          def rrow(i, carry):
            v = 1.0 / jnp.maximum(dback[i, pl.ds(0, 16)], 1.0)
            for k in range(DH // 16):
              zbuf[i, pl.ds(k * 16, 16)] = v
            return carry

          lax.fori_loop(0, 64, rrow, 0)
          pltpu.sync_copy(zbuf, rdeg_hbm.at[pl.ds(r, 64)])

  return agg


_sc_agg_deg = _make_sc_agg(True)
_sc_agg = _make_sc_agg(False)


def _tc_pre(x_ref, w_ref, b_ref, xs_ref, xn_ref):
  y = jnp.dot(x_ref[...], w_ref[...], preferred_element_type=jnp.float32)
  xs_ref[...] = y[:, :D] + b_ref[...]
  xn_ref[...] = y[:, D:]


def _tc_mid(xs_ref, p0_ref, p1_ref, rd_ref, w_ref, b_ref,
            xs2_ref, xn2_ref):
  p0 = p0_ref[0]
  p1 = p1_ref[0]
  rd = rd_ref[...]
  neigh = jnp.concatenate(
      [p0[:, :DH], p1[:, :DH], p0[:, DH:], p1[:, DH:]], axis=1)
  rd2 = jnp.concatenate(
      [rd[:, :DH], rd[:, :DH], rd[:, DH:], rd[:, DH:]], axis=1)
  h1 = jnp.maximum(xs_ref[...] + neigh * rd2, 0.0)
  y = jnp.dot(h1, w_ref[...], preferred_element_type=jnp.float32)
  xs2_ref[...] = jnp.concatenate([y[:, :D], y[:, 2 * D:3 * D]], axis=1) + b_ref[...]
  xn2_ref[...] = jnp.concatenate([y[:, D:2 * D], y[:, 3 * D:]], axis=1)


def _tc_post(xs2_ref, q0_ref, q1_ref, rd_ref, out_ref):
  q0 = q0_ref[0]
  q1 = q1_ref[0]
  rd = rd_ref[...]
  neigh = jnp.concatenate(
      [q0[:, :DH], q1[:, :DH], q0[:, DH:], q1[:, DH:]], axis=1)
  rd2 = jnp.concatenate(
      [rd[:, :DH], rd[:, :DH], rd[:, DH:], rd[:, DH:]], axis=1)
  out_ref[...] = xs2_ref[...] + neigh * rd2


def _row_spec(cols):
  return pl.BlockSpec((_ROWBLK, cols), lambda i: (i, 0))


def _prow_spec(cols):
  return pl.BlockSpec((_PB, cols), lambda i: (i, 0))


def _pair_spec(core):
  return pl.BlockSpec((1, _PB, D), lambda i, c=core: (c, i, 0))


_W_SPEC = pl.BlockSpec((D, 2 * D), lambda i: (0, 0))
_W2_SPEC = pl.BlockSpec((2 * D, 4 * D), lambda i: (0, 0))
_B_SPEC = pl.BlockSpec((1, D), lambda i: (0, 0))
_B2_SPEC = pl.BlockSpec((1, 2 * D), lambda i: (0, 0))


def kernel(x, edge_index, W_self1, W_neigh1, b1, W_self2, W_neigh2, b2):
  eidx = edge_index.reshape(2, NS, NB, B)
  w1 = jnp.concatenate([W_self1, W_neigh1], axis=1)
  w2 = jnp.concatenate([W_self2, W_neigh2], axis=1)
  zer = jnp.zeros((D, 2 * D), jnp.float32)
  w2big = jnp.concatenate(
      [jnp.concatenate([w2, zer], axis=1),
       jnp.concatenate([zer, w2], axis=1)], axis=0)
  b1r = b1.reshape(1, D)
  b2r = jnp.concatenate([b2, b2]).reshape(1, 2 * D)
  onesz = jnp.stack([jnp.ones((B, DG), jnp.float32),
                     jnp.zeros((B, DG), jnp.float32)])

  xs1, xn1 = pl.pallas_call(
      _tc_pre,
      grid=(_GRID,),
      in_specs=[_row_spec(D), _W_SPEC, _B_SPEC],
      out_specs=[_row_spec(D), _row_spec(D)],
      out_shape=[
          jax.ShapeDtypeStruct((N, D), jnp.float32),
          jax.ShapeDtypeStruct((N, D), jnp.float32),
      ],
  )(x, w1, b1r)

  p, rdeg = _sc_agg_deg(xn1.reshape(2 * N, DH), eidx, onesz)
  pp = p.reshape(NC, NP // 2, D)
  rdp = rdeg.reshape(NP // 2, D)
  xs1p = xs1.reshape(N // 2, 2 * D)

  xs2p, xn2p = pl.pallas_call(
      _tc_mid,
      grid=(_GRID,),
      in_specs=[_prow_spec(2 * D), _pair_spec(0), _pair_spec(1),
                _prow_spec(D), _W2_SPEC, _B2_SPEC],
      out_specs=[_prow_spec(2 * D), _prow_spec(2 * D)],
      out_shape=[
          jax.ShapeDtypeStruct((N // 2, 2 * D), jnp.float32),
          jax.ShapeDtypeStruct((N // 2, 2 * D), jnp.float32),
      ],
  )(xs1p, pp, pp, rdp, w2big, b2r)

  q = _sc_agg(xn2p.reshape(2 * N, DH), eidx)
  qq = q.reshape(NC, NP // 2, D)

  outp = pl.pallas_call(
      _tc_post,
      grid=(_GRID,),
      in_specs=[_prow_spec(2 * D), _pair_spec(0), _pair_spec(1),
                _prow_spec(D)],
      out_specs=_prow_spec(2 * D),
      out_shape=jax.ShapeDtypeStruct((N // 2, 2 * D), jnp.float32),
  )(xs2p, qq, qq, rdp)

  return outp.reshape(N, D)


# speedup vs baseline: 1.0835x; 1.0392x over previous
import functools

import jax
import jax.numpy as jnp
from jax import lax
from jax.experimental import pallas as pl
from jax.experimental.pallas import tpu as pltpu
from jax.experimental.pallas import tpu_sc as plsc

N = 10000
E = 320000
D = 128
DH = 64
DG = 16

NC = 2
NS = 16
B = 80
EPT = E // NS
NB = EPT // B
NP = 10240
S = 6
F = 4

_ROWBLK = 2000
_GRID = N // _ROWBLK
_PB = _ROWBLK // 2


def _make_sc_agg(with_deg):
  mesh = plsc.VectorSubcoreMesh(
      core_axis_name="c", subcore_axis_name="s", num_cores=NC, num_subcores=NS)

  out_type = [jax.ShapeDtypeStruct((NC, NP, DH), jnp.float32)]
  scratch = [
      pltpu.VMEM((NB, B), jnp.int32),
      pltpu.VMEM((NB, B), jnp.int32),
      [pltpu.VMEM((B, DH), jnp.float32) for _ in range(S)],
      [pltpu.SemaphoreType.DMA for _ in range(S)],
      [pltpu.SemaphoreType.DMA for _ in range(S)],
      pltpu.VMEM((64, DH), jnp.float32),
      pltpu.VMEM_SHARED((NP, DH), jnp.float32),
  ]
  if with_deg:
    out_type.append(jax.ShapeDtypeStruct((NC, NP, DH), jnp.float32))
    scratch += [
        pltpu.VMEM((B, DG), jnp.float32),
        pltpu.VMEM((64, DG), jnp.float32),
        pltpu.VMEM((64, DG), jnp.float32),
        pltpu.VMEM_SHARED((NP, DG), jnp.float32),
        pltpu.SemaphoreType.DMA,
    ]

  @functools.partial(
      pl.kernel,
      out_type=tuple(out_type) if with_deg else out_type[0],
      mesh=mesh,
      compiler_params=pltpu.CompilerParams(use_tc_tiling_on_sc=False),
      scratch_types=scratch,
  )
  def agg(*refs):
    if with_deg:
      (h2_hbm, eidx_hbm, onesz_hbm, out_hbm, dbc_hbm,
       src_v, dst_v, rows, gsems, ssems, zbuf, acc,
       ones_v, zdeg_v, dback, degacc, dsem) = refs
    else:
      (h2_hbm, eidx_hbm, out_hbm,
       src_v, dst_v, rows, gsems, ssems, zbuf, acc) = refs

    c = lax.axis_index("c")
    s = lax.axis_index("s")

    z16 = jnp.zeros((16,), jnp.float32)

    def zrow(i, carry):
      for j in range(DH // 16):
        zbuf[i, pl.ds(j * 16, 16)] = z16
      return carry

    lax.fori_loop(0, 64, zrow, 0)
    if with_deg:
      pltpu.sync_copy(onesz_hbm.at[0], ones_v)
      pltpu.sync_copy(onesz_hbm.at[1, pl.ds(0, 64)], zdeg_v)
    for j in range(10):
      r = s * 640 + j * 64
      pltpu.sync_copy(zbuf, acc.at[pl.ds(r, 64)])
      if with_deg:
        pltpu.sync_copy(zdeg_v, degacc.at[pl.ds(r, 64)])
    plsc.subcore_barrier()

    pltpu.sync_copy(eidx_hbm.at[0, s], src_v)
    pltpu.sync_copy(eidx_hbm.at[1, s], dst_v)

    def tx(i, carry):
      for j in range(B // 16):
        sl = (i, pl.ds(j * 16, 16))
        src_v[sl] = src_v[sl] * 2 + c
      return carry

    lax.fori_loop(0, NB, tx, 0)

    if with_deg:
      dlo = c * (NB // 2)
      dhi = (c + 1) * (NB // 2)

    def visit(i, r, guard_ssem):
      r2 = (r + F) % S
      wait_sc = lambda: pltpu.make_async_copy(
          rows[r2], acc.at[dst_v.at[i + F - S]], ssems[r2]).wait()
      if guard_ssem:
        pl.when(i + F - S >= 0)(wait_sc)
      else:
        wait_sc()
      pltpu.async_copy(h2_hbm.at[src_v.at[i + F]], rows[r2], gsems[r2])
      pltpu.make_async_copy(h2_hbm.at[src_v.at[i]], rows[r], gsems[r]).wait()
      pltpu.async_copy(rows[r], acc.at[dst_v.at[i]], ssems[r], add=True)
      if with_deg:
        def _fire_deg(i=i):
          pltpu.async_copy(ones_v, degacc.at[dst_v.at[i]], dsem, add=True)

        pl.when((i >= dlo) & (i < dhi))(_fire_deg)

    for j in range(F):
      pltpu.async_copy(h2_hbm.at[src_v.at[j]], rows[j], gsems[j])

    nloop = (NB - F) // S

    def body(g, carry):
      for r in range(S):
        visit(S * g + r, r, guard_ssem=True)
      return carry

    lax.fori_loop(0, nloop, body, 0)

    for i in range(S * nloop, NB):
      r = i % S
      pltpu.make_async_copy(h2_hbm.at[src_v.at[i]], rows[r], gsems[r]).wait()
      pltpu.async_copy(rows[r], acc.at[dst_v.at[i]], ssems[r], add=True)
      if with_deg:
        def _fire_deg(i=i):
          pltpu.async_copy(ones_v, degacc.at[dst_v.at[i]], dsem, add=True)

        pl.when((i >= dlo) & (i < dhi))(_fire_deg)

    for j in range(S * nloop + F - S, NB):
      rj = j % S
      pltpu.make_async_copy(rows[rj], acc.at[dst_v.at[j]], ssems[rj]).wait()

    if with_deg:
      def ddrain(i, carry):
        pltpu.make_async_copy(ones_v, degacc.at[dst_v.at[i]], dsem).wait()
        return carry

      lax.fori_loop(dlo, dhi, ddrain, 0)

    plsc.subcore_barrier()

    for j in range(10):
      r = s * 640 + j * 64
      pltpu.sync_copy(acc.at[pl.ds(r, 64)], out_hbm.at[c, pl.ds(r, 64)])
      if with_deg:
        pltpu.sync_copy(degacc.at[pl.ds(r, 64)], dback)

        def rrow(i, carry):
          v = dback[i, pl.ds(0, 16)]
          for k in range(DH // 16):
            zbuf[i, pl.ds(k * 16, 16)] = v
          return carry

        lax.fori_loop(0, 64, rrow, 0)
        pltpu.sync_copy(zbuf, dbc_hbm.at[c, pl.ds(r, 64)])

  return agg


_sc_agg_deg = _make_sc_agg(True)
_sc_agg = _make_sc_agg(False)


def _tc_pre(x_ref, w_ref, b_ref, xs_ref, xn_ref):
  y = jnp.dot(x_ref[...], w_ref[...], preferred_element_type=jnp.float32)
  xs_ref[...] = y[:, :D] + b_ref[...]
  xn_ref[...] = y[:, D:]


def _tc_mid(xs_ref, p0_ref, p1_ref, d0_ref, d1_ref, w_ref, b_ref,
            xs2_ref, xn2_ref):
  p0 = p0_ref[0]
  p1 = p1_ref[0]
  rd = 1.0 / jnp.maximum(d0_ref[0] + d1_ref[0], 1.0)
  neigh = jnp.concatenate(
      [p0[:, :DH], p1[:, :DH], p0[:, DH:], p1[:, DH:]], axis=1)
  rd2 = jnp.concatenate(
      [rd[:, :DH], rd[:, :DH], rd[:, DH:], rd[:, DH:]], axis=1)
  h1 = jnp.maximum(xs_ref[...] + neigh * rd2, 0.0)
  y = jnp.dot(h1, w_ref[...], preferred_element_type=jnp.float32)
  xs2_ref[...] = jnp.concatenate([y[:, :D], y[:, 2 * D:3 * D]], axis=1) + b_ref[...]
  xn2_ref[...] = jnp.concatenate([y[:, D:2 * D], y[:, 3 * D:]], axis=1)


def _tc_post(xs2_ref, q0_ref, q1_ref, d0_ref, d1_ref, out_ref):
  q0 = q0_ref[0]
  q1 = q1_ref[0]
  rd = 1.0 / jnp.maximum(d0_ref[0] + d1_ref[0], 1.0)
  neigh = jnp.concatenate(
      [q0[:, :DH], q1[:, :DH], q0[:, DH:], q1[:, DH:]], axis=1)
  rd2 = jnp.concatenate(
      [rd[:, :DH], rd[:, :DH], rd[:, DH:], rd[:, DH:]], axis=1)
  out_ref[...] = xs2_ref[...] + neigh * rd2


def _row_spec(cols):
  return pl.BlockSpec((_ROWBLK, cols), lambda i: (i, 0))


def _prow_spec(cols):
  return pl.BlockSpec((_PB, cols), lambda i: (i, 0))


def _pair_spec(core):
  return pl.BlockSpec((1, _PB, D), lambda i, c=core: (c, i, 0))


_W_SPEC = pl.BlockSpec((D, 2 * D), lambda i: (0, 0))
_W2_SPEC = pl.BlockSpec((2 * D, 4 * D), lambda i: (0, 0))
_B_SPEC = pl.BlockSpec((1, D), lambda i: (0, 0))
_B2_SPEC = pl.BlockSpec((1, 2 * D), lambda i: (0, 0))


def kernel(x, edge_index, W_self1, W_neigh1, b1, W_self2, W_neigh2, b2):
  eidx = edge_index.reshape(2, NS, NB, B)
  w1 = jnp.concatenate([W_self1, W_neigh1], axis=1)
  w2 = jnp.concatenate([W_self2, W_neigh2], axis=1)
  zer = jnp.zeros((D, 2 * D), jnp.float32)
  w2big = jnp.concatenate(
      [jnp.concatenate([w2, zer], axis=1),
       jnp.concatenate([zer, w2], axis=1)], axis=0)
  b1r = b1.reshape(1, D)
  b2r = jnp.concatenate([b2, b2]).reshape(1, 2 * D)
  onesz = jnp.stack([jnp.ones((B, DG), jnp.float32),
                     jnp.zeros((B, DG), jnp.float32)])

  xs1, xn1 = pl.pallas_call(
      _tc_pre,
      grid=(_GRID,),
      in_specs=[_row_spec(D), _W_SPEC, _B_SPEC],
      out_specs=[_row_spec(D), _row_spec(D)],
      out_shape=[
          jax.ShapeDtypeStruct((N, D), jnp.float32),
          jax.ShapeDtypeStruct((N, D), jnp.float32),
      ],
  )(x, w1, b1r)

  p, dbc = _sc_agg_deg(xn1.reshape(2 * N, DH), eidx, onesz)
  pp = p.reshape(NC, NP // 2, D)
  dbp = dbc.reshape(NC, NP // 2, D)
  xs1p = xs1.reshape(N // 2, 2 * D)

  xs2p, xn2p = pl.pallas_call(
      _tc_mid,
      grid=(_GRID,),
      in_specs=[_prow_spec(2 * D), _pair_spec(0), _pair_spec(1),
                _pair_spec(0), _pair_spec(1), _W2_SPEC, _B2_SPEC],
      out_specs=[_prow_spec(2 * D), _prow_spec(2 * D)],
      out_shape=[
          jax.ShapeDtypeStruct((N // 2, 2 * D), jnp.float32),
          jax.ShapeDtypeStruct((N // 2, 2 * D), jnp.float32),
      ],
  )(xs1p, pp, pp, dbp, dbp, w2big, b2r)

  q = _sc_agg(xn2p.reshape(2 * N, DH), eidx)
  qq = q.reshape(NC, NP // 2, D)

  outp = pl.pallas_call(
      _tc_post,
      grid=(_GRID,),
      in_specs=[_prow_spec(2 * D), _pair_spec(0), _pair_spec(1),
                _pair_spec(0), _pair_spec(1)],
      out_specs=_prow_spec(2 * D),
      out_shape=jax.ShapeDtypeStruct((N // 2, 2 * D), jnp.float32),
  )(xs2p, qq, qq, dbp, dbp)

  return outp.reshape(N, D)
